# baseline (speedup 1.0000x reference)
import jax
import jax.numpy as jnp
from jax.experimental import pallas as pl
from jax.experimental.pallas import tpu as pltpu

NUM_BLOCKS = 4096
BLOCK_SIZE = 16
NUM_KV_HEADS = 8
HEAD_DIM = 128
NUM_TOKENS = 16384


def _copy_body(c_ref, o_ref):
    o_ref[...] = c_ref[...]


def _scatter_body(bi_ref, bo_ref, in_ref, old_ref, out_ref):
    out_ref[0, 0] = in_ref[0]


def kernel(input, cache, block_indices, block_offset):
    blk = 32
    copied = pl.pallas_call(
        _copy_body,
        grid=(NUM_BLOCKS // blk,),
        in_specs=[pl.BlockSpec((blk, BLOCK_SIZE, NUM_KV_HEADS, HEAD_DIM),
                               lambda i: (i, 0, 0, 0))],
        out_specs=pl.BlockSpec((blk, BLOCK_SIZE, NUM_KV_HEADS, HEAD_DIM),
                               lambda i: (i, 0, 0, 0)),
        out_shape=jax.ShapeDtypeStruct(cache.shape, cache.dtype),
    )(cache)

    grid_spec = pltpu.PrefetchScalarGridSpec(
        num_scalar_prefetch=2,
        grid=(NUM_TOKENS,),
        in_specs=[
            pl.BlockSpec((1, NUM_KV_HEADS, HEAD_DIM), lambda i, bi, bo: (i, 0, 0)),
            pl.BlockSpec(memory_space=pl.ANY),
        ],
        out_specs=pl.BlockSpec((1, 1, NUM_KV_HEADS, HEAD_DIM),
                               lambda i, bi, bo: (bi[i], bo[i], 0, 0)),
    )
    out = pl.pallas_call(
        _scatter_body,
        grid_spec=grid_spec,
        out_shape=jax.ShapeDtypeStruct(cache.shape, cache.dtype),
        input_output_aliases={3: 0},
    )(block_indices, block_offset, input, copied)
    return out


# speedup vs baseline: 9.4774x; 9.4774x over previous
import functools

import jax
import jax.numpy as jnp
from jax import lax
from jax.experimental import pallas as pl
from jax.experimental.pallas import tpu as pltpu
from jax.experimental.pallas import tpu_sc as plsc

NUM_BLOCKS = 4096
BLOCK_SIZE = 16
NUM_KV_HEADS = 8
HEAD_DIM = 128
NUM_TOKENS = 16384

NUM_SLOTS = NUM_BLOCKS * BLOCK_SIZE
ROW = NUM_KV_HEADS * HEAD_DIM
NUM_WORKERS = 32
SLOTS_PER_W = NUM_SLOTS // NUM_WORKERS
L = 16
CCHUNK = 32
NGROUPS_MAX = SLOTS_PER_W // L


def _dyn_gather(v, idx):
    dnums = lax.GatherDimensionNumbers(
        offset_dims=(), collapsed_slice_dims=(0,), start_index_map=(0,))
    return lax.gather(v, idx[:, None], dnums, (1,),
                      mode=lax.GatherScatterMode.PROMISE_IN_BOUNDS)


def _sc_body(inp_hbm, cache_hbm, bi_hbm, bo_hbm, out_hbm,
             bi_v, bo_v, winner_v, tids_v, slots_v, cbuf_v, rows_v,
             c_sem, g_sem, s_sem):
    c = lax.axis_index("c")
    s = lax.axis_index("s")
    w = s * 2 + c
    base_slot = w * SLOTS_PER_W

    pltpu.sync_copy(bi_hbm, bi_v)
    pltpu.sync_copy(bo_hbm, bo_v)

    iota = lax.iota(jnp.int32, L)
    neg1 = jnp.full((L,), -1, jnp.int32)

    def init_body(j, carry):
        winner_v[pl.ds(j * L, L)] = neg1
        return carry
    lax.fori_loop(0, SLOTS_PER_W // L, init_body, 0)

    def scan_body(i, carry):
        bi16 = bi_v[pl.ds(i * L, L)]
        bo16 = bo_v[pl.ds(i * L, L)]
        slot = bi16 * BLOCK_SIZE + bo16
        tid = i * L + iota
        own = jnp.logical_and(slot >= base_slot,
                              slot < base_slot + SLOTS_PER_W)
        _, last = plsc.scan_count(slot)
        m = jnp.logical_and(own, last)
        idx = jnp.clip(slot - base_slot, 0, SLOTS_PER_W - 1)
        plsc.store_scatter(winner_v, [idx], tid, mask=m)
        return carry
    lax.fori_loop(0, NUM_TOKENS // L, scan_body, 0)

    fifteen = jnp.full((L,), L - 1, jnp.int32)

    def comp_body(j, carry):
        nwv, tid_pad, slot_pad = carry
        wv = winner_v[pl.ds(j * L, L)]
        m = wv >= 0
        pc = plsc.cumsum(jnp.where(m, 1, 0))
        pos = jnp.clip(nwv + pc - 1, 0, SLOTS_PER_W - 1)
        slots16 = base_slot + j * L + iota
        plsc.store_scatter(tids_v, [pos >> 4, pos & (L - 1)], wv, mask=m)
        plsc.store_scatter(slots_v, [pos >> 4, pos & (L - 1)], slots16,
                           mask=m)
        cnt = plsc.all_reduce_population_count(m)
        lstar = _dyn_gather(plsc.cummax(jnp.where(m, iota, -1)), fifteen)
        has = cnt > 0
        lstar_c = jnp.clip(lstar, 0, L - 1)
        tid_pad = jnp.where(has, _dyn_gather(wv, lstar_c), tid_pad)
        slot_pad = jnp.where(has, _dyn_gather(slots16, lstar_c), slot_pad)
        return nwv + cnt, tid_pad, slot_pad

    nwv, tid_pad, slot_pad = lax.fori_loop(
        0, SLOTS_PER_W // L, comp_body,
        (jnp.zeros((L,), jnp.int32), jnp.zeros((L,), jnp.int32),
         base_slot + jnp.zeros((L,), jnp.int32)))
    nw = jnp.clip(jnp.max(nwv), 0, SLOTS_PER_W)

    tpos = jnp.clip(nw + iota, 0, SLOTS_PER_W + L - 1)
    plsc.store_scatter(tids_v, [tpos >> 4, tpos & (L - 1)], tid_pad)
    plsc.store_scatter(slots_v, [tpos >> 4, tpos & (L - 1)], slot_pad)

    def copy_body(k, carry):
        src = cache_hbm.at[pl.ds(base_slot + k * CCHUNK, CCHUNK)]
        dst = out_hbm.at[pl.ds(base_slot + k * CCHUNK, CCHUNK)]
        pltpu.sync_copy(src, cbuf_v)
        pltpu.sync_copy(cbuf_v, dst)
        return carry
    lax.fori_loop(0, SLOTS_PER_W // CCHUNK, copy_body, 0)

    ngroups = jnp.clip(lax.div(nw + L - 1, L), 0, NGROUPS_MAX)

    def scat_body(g, carry):
        gcp = pltpu.make_async_copy(
            inp_hbm.at[tids_v.at[g]], rows_v, g_sem)
        gcp.start()
        gcp.wait()
        scp = pltpu.make_async_copy(
            rows_v, out_hbm.at[slots_v.at[g]], s_sem)
        scp.start()
        scp.wait()
        return carry
    lax.fori_loop(0, ngroups, scat_body, 0)


def kernel(input, cache, block_indices, block_offset):
    inp2 = input.reshape(NUM_TOKENS, ROW)
    cache2 = cache.reshape(NUM_SLOTS, ROW)
    mesh = plsc.VectorSubcoreMesh(core_axis_name="c", subcore_axis_name="s")
    run = functools.partial(
        pl.kernel,
        out_type=jax.ShapeDtypeStruct((NUM_SLOTS, ROW), jnp.float32),
        mesh=mesh,
        compiler_params=pltpu.CompilerParams(needs_layout_passes=False),
        scratch_types=[
            pltpu.VMEM((NUM_TOKENS,), jnp.int32),
            pltpu.VMEM((NUM_TOKENS,), jnp.int32),
            pltpu.VMEM((SLOTS_PER_W,), jnp.int32),
            pltpu.VMEM((NGROUPS_MAX + 1, L), jnp.int32),
            pltpu.VMEM((NGROUPS_MAX + 1, L), jnp.int32),
            pltpu.VMEM((CCHUNK, ROW), jnp.float32),
            pltpu.VMEM((L, ROW), jnp.float32),
            pltpu.SemaphoreType.DMA,
            pltpu.SemaphoreType.DMA,
            pltpu.SemaphoreType.DMA,
        ],
    )(_sc_body)
    out = run(inp2, cache2, block_indices, block_offset)
    return out.reshape(cache.shape)


# speedup vs baseline: 10.5525x; 1.1134x over previous
import functools

import jax
import jax.numpy as jnp
from jax import lax
from jax.experimental import pallas as pl
from jax.experimental.pallas import tpu as pltpu
from jax.experimental.pallas import tpu_sc as plsc

NUM_BLOCKS = 4096
BLOCK_SIZE = 16
NUM_KV_HEADS = 8
HEAD_DIM = 128
NUM_TOKENS = 16384

NUM_SLOTS = NUM_BLOCKS * BLOCK_SIZE
ROW = NUM_KV_HEADS * HEAD_DIM
NUM_WORKERS = 32
SLOTS_PER_W = NUM_SLOTS // NUM_WORKERS
L = 16
CCHUNK = 16
NCHUNKS = SLOTS_PER_W // CCHUNK
NBUF = 2
NITER = NCHUNKS // NBUF
NGROUPS_MAX = SLOTS_PER_W // L


def _dyn_gather(v, idx):
    dnums = lax.GatherDimensionNumbers(
        offset_dims=(), collapsed_slice_dims=(0,), start_index_map=(0,))
    return lax.gather(v, idx[:, None], dnums, (1,),
                      mode=lax.GatherScatterMode.PROMISE_IN_BOUNDS)


def _sc_body(inp_hbm, cache_hbm, bi_hbm, bo_hbm, out_hbm,
             bi_v, bo_v, winner_v, tids_v, slots_v, rows_v, cshared,
             in_sems, out_sems, g_sem, s_sem):
    c = lax.axis_index("c")
    s = lax.axis_index("s")
    w = s * 2 + c
    base_slot = w * SLOTS_PER_W
    base_chunk = base_slot // CCHUNK

    for b in range(NBUF):
        pltpu.make_async_copy(
            cache_hbm.at[pl.ds((base_chunk + b) * CCHUNK, CCHUNK)],
            cshared.at[s, b], in_sems.at[b]).start()

    pltpu.sync_copy(bi_hbm, bi_v)
    pltpu.sync_copy(bo_hbm, bo_v)

    iota = lax.iota(jnp.int32, L)
    neg1 = jnp.full((L,), -1, jnp.int32)

    def init_body(j, carry):
        winner_v[pl.ds(j * L, L)] = neg1
        return carry
    lax.fori_loop(0, SLOTS_PER_W // L, init_body, 0)

    ngroup_per_iter = (NUM_TOKENS // L) // NITER

    def scan_group(i):
        bi16 = bi_v[pl.ds(i * L, L)]
        bo16 = bo_v[pl.ds(i * L, L)]
        slot = bi16 * BLOCK_SIZE + bo16
        tid = i * L + iota
        own = jnp.logical_and(slot >= base_slot,
                              slot < base_slot + SLOTS_PER_W)
        _, last = plsc.scan_count(slot)
        m = jnp.logical_and(own, last)
        idx = jnp.clip(slot - base_slot, 0, SLOTS_PER_W - 1)
        plsc.store_scatter(winner_v, [idx], tid, mask=m)

    def fused_body(kk, carry):
        for t in range(ngroup_per_iter // 2):
            scan_group(kk * ngroup_per_iter + t)
        for b in range(NBUF):
            ch = kk * NBUF + b
            pltpu.make_async_copy(
                cache_hbm.at[pl.ds((base_chunk + ch) * CCHUNK, CCHUNK)],
                cshared.at[s, b], in_sems.at[b]).wait()
            pltpu.make_async_copy(
                cshared.at[s, b],
                out_hbm.at[pl.ds((base_chunk + ch) * CCHUNK, CCHUNK)],
                out_sems.at[b]).start()
        for t in range(ngroup_per_iter // 2, ngroup_per_iter):
            scan_group(kk * ngroup_per_iter + t)
        for b in range(NBUF):
            ch = jnp.minimum(kk * NBUF + b + NBUF, NCHUNKS - 1)
            pltpu.make_async_copy(
                cshared.at[s, b],
                out_hbm.at[pl.ds((base_chunk + ch) * CCHUNK, CCHUNK)],
                out_sems.at[b]).wait()
            pltpu.make_async_copy(
                cache_hbm.at[pl.ds((base_chunk + ch) * CCHUNK, CCHUNK)],
                cshared.at[s, b], in_sems.at[b]).start()
        return carry
    lax.fori_loop(0, NITER, fused_body, 0)

    for b in range(NBUF):
        ch = NCHUNKS - 1
        pltpu.make_async_copy(
            cache_hbm.at[pl.ds((base_chunk + ch) * CCHUNK, CCHUNK)],
            cshared.at[s, b], in_sems.at[b]).wait()

    fifteen = jnp.full((L,), L - 1, jnp.int32)

    def comp_body(j, carry):
        nwv, tid_pad, slot_pad = carry
        wv = winner_v[pl.ds(j * L, L)]
        m = wv >= 0
        pc = plsc.cumsum(jnp.where(m, 1, 0))
        pos = jnp.clip(nwv + pc - 1, 0, SLOTS_PER_W - 1)
        slots16 = base_slot + j * L + iota
        plsc.store_scatter(tids_v, [pos >> 4, pos & (L - 1)], wv, mask=m)
        plsc.store_scatter(slots_v, [pos >> 4, pos & (L - 1)], slots16,
                           mask=m)
        cnt = plsc.all_reduce_population_count(m)
        lstar = _dyn_gather(plsc.cummax(jnp.where(m, iota, -1)), fifteen)
        has = cnt > 0
        lstar_c = jnp.clip(lstar, 0, L - 1)
        tid_pad = jnp.where(has, _dyn_gather(wv, lstar_c), tid_pad)
        slot_pad = jnp.where(has, _dyn_gather(slots16, lstar_c), slot_pad)
        return nwv + cnt, tid_pad, slot_pad

    nwv, tid_pad, slot_pad = lax.fori_loop(
        0, SLOTS_PER_W // L, comp_body,
        (jnp.zeros((L,), jnp.int32), jnp.zeros((L,), jnp.int32),
         base_slot + jnp.zeros((L,), jnp.int32)))
    nw = jnp.clip(jnp.max(nwv), 0, SLOTS_PER_W)

    tpos = jnp.clip(nw + iota, 0, SLOTS_PER_W + L - 1)
    plsc.store_scatter(tids_v, [tpos >> 4, tpos & (L - 1)], tid_pad)
    plsc.store_scatter(slots_v, [tpos >> 4, tpos & (L - 1)], slot_pad)

    ngroups = jnp.clip(lax.div(nw + L - 1, L), 0, NGROUPS_MAX)

    def scat_body(g, carry):
        gcp = pltpu.make_async_copy(
            inp_hbm.at[tids_v.at[g]], rows_v, g_sem)
        gcp.start()
        gcp.wait()
        scp = pltpu.make_async_copy(
            rows_v, out_hbm.at[slots_v.at[g]], s_sem)
        scp.start()
        scp.wait()
        return carry
    lax.fori_loop(0, ngroups, scat_body, 0)


def kernel(input, cache, block_indices, block_offset):
    inp2 = input.reshape(NUM_TOKENS, ROW)
    cache2 = cache.reshape(NUM_SLOTS, ROW)
    mesh = plsc.VectorSubcoreMesh(core_axis_name="c", subcore_axis_name="s")
    run = functools.partial(
        pl.kernel,
        out_type=jax.ShapeDtypeStruct((NUM_SLOTS, ROW), jnp.float32),
        mesh=mesh,
        compiler_params=pltpu.CompilerParams(needs_layout_passes=False),
        scratch_types=[
            pltpu.VMEM((NUM_TOKENS,), jnp.int32),
            pltpu.VMEM((NUM_TOKENS,), jnp.int32),
            pltpu.VMEM((SLOTS_PER_W,), jnp.int32),
            pltpu.VMEM((NGROUPS_MAX + 1, L), jnp.int32),
            pltpu.VMEM((NGROUPS_MAX + 1, L), jnp.int32),
            pltpu.VMEM((L, ROW), jnp.float32),
            pltpu.VMEM_SHARED((16, NBUF, CCHUNK, ROW), jnp.float32),
            pltpu.SemaphoreType.DMA((NBUF,)),
            pltpu.SemaphoreType.DMA((NBUF,)),
            pltpu.SemaphoreType.DMA,
            pltpu.SemaphoreType.DMA,
        ],
    )(_sc_body)
    out = run(inp2, cache2, block_indices, block_offset)
    return out.reshape(cache.shape)


# speedup vs baseline: 10.6666x; 1.0108x over previous
import functools

import jax
import jax.numpy as jnp
from jax import lax
from jax.experimental import pallas as pl
from jax.experimental.pallas import tpu as pltpu
from jax.experimental.pallas import tpu_sc as plsc

NUM_BLOCKS = 4096
BLOCK_SIZE = 16
NUM_KV_HEADS = 8
HEAD_DIM = 128
NUM_TOKENS = 16384

NUM_SLOTS = NUM_BLOCKS * BLOCK_SIZE
ROW = NUM_KV_HEADS * HEAD_DIM
NUM_WORKERS = 32
SLOTS_PER_W = NUM_SLOTS // NUM_WORKERS
L = 16
CCHUNK = 16
NCHUNKS = SLOTS_PER_W // CCHUNK
NBUF = 2
NITER = NCHUNKS // NBUF
SGRP = 32
NSGRP_MAX = SLOTS_PER_W // SGRP


def _dyn_gather(v, idx):
    dnums = lax.GatherDimensionNumbers(
        offset_dims=(), collapsed_slice_dims=(0,), start_index_map=(0,))
    return lax.gather(v, idx[:, None], dnums, (1,),
                      mode=lax.GatherScatterMode.PROMISE_IN_BOUNDS)


def _sc_body(inp_hbm, cache_hbm, bi_hbm, bo_hbm, out_hbm,
             bi_v, bo_v, winner_v, tids_v, slots_v, rows_v, cshared,
             in_sems, out_sems, g_sem, s_sem):
    c = lax.axis_index("c")
    s = lax.axis_index("s")
    w = s * 2 + c
    base_slot = w * SLOTS_PER_W
    base_chunk = base_slot // CCHUNK

    for b in range(NBUF):
        pltpu.make_async_copy(
            cache_hbm.at[pl.ds((base_chunk + b) * CCHUNK, CCHUNK)],
            cshared.at[s, b], in_sems.at[b]).start()

    pltpu.sync_copy(bi_hbm, bi_v)
    pltpu.sync_copy(bo_hbm, bo_v)

    iota = lax.iota(jnp.int32, L)
    neg1 = jnp.full((L,), -1, jnp.int32)

    def init_body(j, carry):
        winner_v[pl.ds(j * L, L)] = neg1
        return carry
    lax.fori_loop(0, SLOTS_PER_W // L, init_body, 0)

    ngroup_per_iter = (NUM_TOKENS // L) // NITER

    def scan_group(i):
        bi16 = bi_v[pl.ds(i * L, L)]
        bo16 = bo_v[pl.ds(i * L, L)]
        slot = bi16 * BLOCK_SIZE + bo16
        tid = i * L + iota
        own = jnp.logical_and(slot >= base_slot,
                              slot < base_slot + SLOTS_PER_W)
        _, last = plsc.scan_count(slot)
        m = jnp.logical_and(own, last)
        idx = jnp.clip(slot - base_slot, 0, SLOTS_PER_W - 1)
        plsc.store_scatter(winner_v, [idx], tid, mask=m)

    def fused_body(kk, carry):
        for t in range(ngroup_per_iter // 2):
            scan_group(kk * ngroup_per_iter + t)
        for b in range(NBUF):
            ch = kk * NBUF + b
            pltpu.make_async_copy(
                cache_hbm.at[pl.ds((base_chunk + ch) * CCHUNK, CCHUNK)],
                cshared.at[s, b], in_sems.at[b]).wait()
            pltpu.make_async_copy(
                cshared.at[s, b],
                out_hbm.at[pl.ds((base_chunk + ch) * CCHUNK, CCHUNK)],
                out_sems.at[b]).start()
        for t in range(ngroup_per_iter // 2, ngroup_per_iter):
            scan_group(kk * ngroup_per_iter + t)
        for b in range(NBUF):
            ch = jnp.minimum(kk * NBUF + b + NBUF, NCHUNKS - 1)
            pltpu.make_async_copy(
                cshared.at[s, b],
                out_hbm.at[pl.ds((base_chunk + ch) * CCHUNK, CCHUNK)],
                out_sems.at[b]).wait()
            pltpu.make_async_copy(
                cache_hbm.at[pl.ds((base_chunk + ch) * CCHUNK, CCHUNK)],
                cshared.at[s, b], in_sems.at[b]).start()
        return carry
    lax.fori_loop(0, NITER, fused_body, 0)

    for b in range(NBUF):
        ch = NCHUNKS - 1
        pltpu.make_async_copy(
            cache_hbm.at[pl.ds((base_chunk + ch) * CCHUNK, CCHUNK)],
            cshared.at[s, b], in_sems.at[b]).wait()

    fifteen = jnp.full((L,), L - 1, jnp.int32)

    def comp_body(j, carry):
        nwv, tid_pad, slot_pad = carry
        wv = winner_v[pl.ds(j * L, L)]
        m = wv >= 0
        pc = plsc.cumsum(jnp.where(m, 1, 0))
        pos = jnp.clip(nwv + pc - 1, 0, SLOTS_PER_W - 1)
        slots16 = base_slot + j * L + iota
        plsc.store_scatter(tids_v, [pos >> 5, pos & (SGRP - 1)], wv, mask=m)
        plsc.store_scatter(slots_v, [pos >> 5, pos & (SGRP - 1)], slots16,
                           mask=m)
        cnt = plsc.all_reduce_population_count(m)
        lstar = _dyn_gather(plsc.cummax(jnp.where(m, iota, -1)), fifteen)
        has = cnt > 0
        lstar_c = jnp.clip(lstar, 0, L - 1)
        tid_pad = jnp.where(has, _dyn_gather(wv, lstar_c), tid_pad)
        slot_pad = jnp.where(has, _dyn_gather(slots16, lstar_c), slot_pad)
        return nwv + cnt, tid_pad, slot_pad

    nwv, tid_pad, slot_pad = lax.fori_loop(
        0, SLOTS_PER_W // L, comp_body,
        (jnp.zeros((L,), jnp.int32), jnp.zeros((L,), jnp.int32),
         base_slot + jnp.zeros((L,), jnp.int32)))
    nw = jnp.clip(jnp.max(nwv), 0, SLOTS_PER_W)

    for q in range(2):
        tpos = jnp.clip(nw + q * L + iota, 0, SLOTS_PER_W + SGRP - 1)
        plsc.store_scatter(tids_v, [tpos >> 5, tpos & (SGRP - 1)], tid_pad)
        plsc.store_scatter(slots_v, [tpos >> 5, tpos & (SGRP - 1)],
                           slot_pad)

    ngroups = jnp.clip(lax.div(nw + SGRP - 1, SGRP), 0, NSGRP_MAX)

    def scat_body(g, carry):
        gcp = pltpu.make_async_copy(
            inp_hbm.at[tids_v.at[g]], rows_v, g_sem)
        gcp.start()
        gcp.wait()
        scp = pltpu.make_async_copy(
            rows_v, out_hbm.at[slots_v.at[g]], s_sem)
        scp.start()
        scp.wait()
        return carry
    lax.fori_loop(0, ngroups, scat_body, 0)


def kernel(input, cache, block_indices, block_offset):
    inp2 = input.reshape(NUM_TOKENS, ROW)
    cache2 = cache.reshape(NUM_SLOTS, ROW)
    mesh = plsc.VectorSubcoreMesh(core_axis_name="c", subcore_axis_name="s")
    run = functools.partial(
        pl.kernel,
        out_type=jax.ShapeDtypeStruct((NUM_SLOTS, ROW), jnp.float32),
        mesh=mesh,
        compiler_params=pltpu.CompilerParams(needs_layout_passes=False),
        scratch_types=[
            pltpu.VMEM((NUM_TOKENS,), jnp.int32),
            pltpu.VMEM((NUM_TOKENS,), jnp.int32),
            pltpu.VMEM((SLOTS_PER_W,), jnp.int32),
            pltpu.VMEM((NSGRP_MAX + 1, SGRP), jnp.int32),
            pltpu.VMEM((NSGRP_MAX + 1, SGRP), jnp.int32),
            pltpu.VMEM((SGRP, ROW), jnp.float32),
            pltpu.VMEM_SHARED((16, NBUF, CCHUNK, ROW), jnp.float32),
            pltpu.SemaphoreType.DMA((NBUF,)),
            pltpu.SemaphoreType.DMA((NBUF,)),
            pltpu.SemaphoreType.DMA,
            pltpu.SemaphoreType.DMA,
        ],
    )(_sc_body)
    out = run(inp2, cache2, block_indices, block_offset)
    return out.reshape(cache.shape)
